# Initial kernel scaffold; baseline (speedup 1.0000x reference)
import functools

import jax
import jax.numpy as jnp
from jax import lax
from jax.experimental import pallas as pl
from jax.experimental.pallas import tpu as pltpu
from jax.experimental.pallas import tpu_sc as plsc

N = 10000
C = 128
K = 32
F = 64
NP = 10240
NW = 32
PPW = NP // NW
PC = 32
NCHUNK = PPW // PC
GB = PC * K // 128
FV = F // 16

_mesh = plsc.VectorSubcoreMesh(core_axis_name="c", subcore_axis_name="s")


@functools.partial(
    pl.kernel,
    out_type=jax.ShapeDtypeStruct((NP, F), jnp.float32),
    mesh=_mesh,
    scratch_types=[
        pltpu.VMEM((GB, 128), jnp.int32),
        pltpu.VMEM((PC * K, F), jnp.float32),
        pltpu.VMEM((PPW, F), jnp.float32),
        pltpu.SemaphoreType.DMA,
    ],
)
def _gather_max(x2_hbm, idx_hbm, agg_hbm, idx_v, rows_v, out_v, sem):
    wid = lax.axis_index("s") * 2 + lax.axis_index("c")
    idx_row0 = wid * (NCHUNK * GB)

    def chunk_body(cc, carry):
        pltpu.sync_copy(idx_hbm.at[pl.ds(idx_row0 + cc * GB, GB)], idx_v)
        copies = [
            pltpu.async_copy(
                x2_hbm.at[idx_v.at[j]],
                rows_v.at[pl.ds(j * 128, 128)],
                sem,
            )
            for j in range(GB)
        ]
        for cp in copies:
            cp.wait()

        def point_body(p, carry2):
            base = p * K
            accs = [rows_v[base, pl.ds(f * 16, 16)] for f in range(FV)]
            for k in range(1, K):
                for f in range(FV):
                    accs[f] = jnp.maximum(
                        accs[f], rows_v[base + k, pl.ds(f * 16, 16)]
                    )
            row = cc * PC + p
            for f in range(FV):
                out_v[row, pl.ds(f * 16, 16)] = accs[f]
            return carry2

        lax.fori_loop(0, PC, point_body, 0)
        return carry

    lax.fori_loop(0, NCHUNK, chunk_body, 0)
    pltpu.sync_copy(out_v, agg_hbm.at[pl.ds(wid * PPW, PPW)])


BN = 512
NBLK = NP // BN


def _mlp_body(x_ref, agg_ref, w1_ref, b1_ref, w2_ref, b2_ref, out_ref):
    aggb = agg_ref[...]
    h = jnp.maximum(
        jnp.dot(aggb, w1_ref[...], preferred_element_type=jnp.float32)
        + b1_ref[...],
        0.0,
    )
    f = (
        jnp.dot(h, w2_ref[...], preferred_element_type=jnp.float32)
        + b2_ref[...]
    )
    xb = x_ref[...]
    yfull = jnp.concatenate([xb[:, F:], xb[:, :F] + f], axis=1)
    ident = (
        lax.broadcasted_iota(jnp.int32, (BN, BN), 0)
        == lax.broadcasted_iota(jnp.int32, (BN, BN), 1)
    ).astype(jnp.float32)
    out_ref[...] = lax.dot_general(
        yfull,
        ident,
        (((0,), (0,)), ((), ())),
        preferred_element_type=jnp.float32,
    )


_mlp = pl.pallas_call(
    _mlp_body,
    grid=(NBLK,),
    in_specs=[
        pl.BlockSpec((BN, C), lambda i: (i, 0)),
        pl.BlockSpec((BN, F), lambda i: (i, 0)),
        pl.BlockSpec((F, F), lambda i: (0, 0)),
        pl.BlockSpec((1, F), lambda i: (0, 0)),
        pl.BlockSpec((F, F), lambda i: (0, 0)),
        pl.BlockSpec((1, F), lambda i: (0, 0)),
    ],
    out_specs=pl.BlockSpec((C, BN), lambda i: (0, i)),
    out_shape=jax.ShapeDtypeStruct((C, NP), jnp.float32),
)


def kernel(x, neigh_idx, W1, b1, W2, b2):
    x0 = x[0]
    x2c = x0[:, F:]
    x_pad = jnp.pad(x0, ((0, NP - N), (0, 0)))
    idx = neigh_idx[0].astype(jnp.int32)
    idx_pad = jnp.pad(idx, ((0, NP - N), (0, 0)))
    idx_r = idx_pad.reshape(NP * K // 128, 128)
    agg = _gather_max(x2c, idx_r)
    outT = _mlp(x_pad, agg, W1, b1[None], W2, b2[None])
    return outT[None, :, :N]


# speedup vs baseline: 5.6057x; 5.6057x over previous
import functools

import jax
import jax.numpy as jnp
from jax import lax
from jax.experimental import pallas as pl
from jax.experimental.pallas import tpu as pltpu
from jax.experimental.pallas import tpu_sc as plsc

N = 10000
C = 128
K = 32
F = 64
NP = 10240
NW = 32
PPW = NP // NW
PC = 16
NCHUNK = PPW // PC
GB = PC * K // 128
FV = F // 16

_mesh = plsc.VectorSubcoreMesh(
    core_axis_name="c", subcore_axis_name="s", num_cores=2, num_subcores=16
)

_SC_SCRATCH = [
    pltpu.VMEM((GB, 128), jnp.int32),
    pltpu.VMEM((PC * K, C), jnp.float32),
    pltpu.VMEM((PPW, F), jnp.float32),
    pltpu.SemaphoreType.DMA,
]


def _gather_max_body(xt_hbm, idx_hbm, agg_hbm, idx_v, rows_v, out_v, sem):
    wid = lax.axis_index("s") * 2 + lax.axis_index("c")
    idx_row0 = wid * (NCHUNK * GB)

    def chunk_body(cc, carry):
        pltpu.sync_copy(idx_hbm.at[pl.ds(idx_row0 + cc * GB, GB)], idx_v)
        copies = [
            pltpu.async_copy(
                xt_hbm.at[idx_v.at[j]],
                rows_v.at[pl.ds(j * 128, 128)],
                sem,
            )
            for j in range(GB)
        ]
        for cp in copies:
            cp.wait()

        def point_body(p, carry2):
            base = p * K
            accs = [rows_v[base, pl.ds(F + f * 16, 16)] for f in range(FV)]
            for k in range(1, K):
                for f in range(FV):
                    accs[f] = jnp.maximum(
                        accs[f], rows_v[base + k, pl.ds(F + f * 16, 16)]
                    )
            row = cc * PC + p
            for f in range(FV):
                out_v[row, pl.ds(f * 16, 16)] = accs[f]
            return carry2

        lax.fori_loop(0, PC, point_body, 0)
        return carry

    lax.fori_loop(0, NCHUNK, chunk_body, 0)
    pltpu.sync_copy(out_v, agg_hbm.at[pl.ds(wid * PPW, PPW)])


_gather_max = pl.kernel(
    _gather_max_body,
    out_type=jax.ShapeDtypeStruct((NP, F), jnp.float32),
    mesh=_mesh,
    scratch_types=_SC_SCRATCH,
)


BN = 512
NBLK = NP // BN


def _mlp_body(x_ref, agg_ref, w1_ref, b1_ref, w2_ref, b2_ref, out_ref):
    aggb = agg_ref[...]
    h = jnp.maximum(
        jnp.dot(aggb, w1_ref[...], preferred_element_type=jnp.float32)
        + b1_ref[...],
        0.0,
    )
    f = (
        jnp.dot(h, w2_ref[...], preferred_element_type=jnp.float32)
        + b2_ref[...]
    )
    xb = x_ref[...]
    yfull = jnp.concatenate([xb[:, F:], xb[:, :F] + f], axis=1)
    ident = (
        lax.broadcasted_iota(jnp.int32, (BN, BN), 0)
        == lax.broadcasted_iota(jnp.int32, (BN, BN), 1)
    ).astype(jnp.float32)
    out_ref[...] = lax.dot_general(
        yfull,
        ident,
        (((0,), (0,)), ((), ())),
        preferred_element_type=jnp.float32,
    )


_mlp = pl.pallas_call(
    _mlp_body,
    grid=(NBLK,),
    in_specs=[
        pl.BlockSpec((BN, C), lambda i: (i, 0)),
        pl.BlockSpec((BN, F), lambda i: (i, 0)),
        pl.BlockSpec((F, F), lambda i: (0, 0)),
        pl.BlockSpec((1, F), lambda i: (0, 0)),
        pl.BlockSpec((F, F), lambda i: (0, 0)),
        pl.BlockSpec((1, F), lambda i: (0, 0)),
    ],
    out_specs=pl.BlockSpec((C, BN), lambda i: (0, i)),
    out_shape=jax.ShapeDtypeStruct((C, NP), jnp.float32),
)


def kernel(x, neigh_idx, W1, b1, W2, b2):
    x0 = x[0]
    x_pad = jnp.pad(x0, ((0, NP - N), (0, 0)))
    idx = neigh_idx[0].astype(jnp.int32)
    idx_pad = jnp.pad(idx, ((0, NP - N), (0, 0)))
    idx_r = idx_pad.reshape(NP * K // 128, 128)
    agg = _gather_max(x0, idx_r)
    outT = _mlp(x_pad, agg, W1, b1[None], W2, b2[None])
    return outT[None, :, :N]


# speedup vs baseline: 5.8431x; 1.0423x over previous
import functools

import jax
import jax.numpy as jnp
from jax import lax
from jax.experimental import pallas as pl
from jax.experimental.pallas import tpu as pltpu
from jax.experimental.pallas import tpu_sc as plsc

N = 10000
C = 128
K = 32
F = 64
NP = 10240
NW = 32
PPW = NP // NW
PC = 8
NCHUNK = PPW // PC
GB = PC * K // 128
IPW = NCHUNK * GB
FV = F // 16

_mesh = plsc.VectorSubcoreMesh(
    core_axis_name="c", subcore_axis_name="s", num_cores=2, num_subcores=16
)

_SC_SCRATCH = [
    pltpu.VMEM((IPW, 128), jnp.int32),
    pltpu.VMEM((2, PC * K, C), jnp.float32),
    pltpu.VMEM((PPW, F), jnp.float32),
    pltpu.SemaphoreType.DMA,
    pltpu.SemaphoreType.DMA,
]


def _gather_max_body(xt_hbm, idx_hbm, agg_hbm, idx_v, rows_v, out_v, sem0, sem1):
    wid = lax.axis_index("s") * 2 + lax.axis_index("c")
    sems = (sem0, sem1)

    pltpu.sync_copy(idx_hbm.at[pl.ds(wid * IPW, IPW)], idx_v)

    def fire(cc, b):
        for j in range(GB):
            pltpu.async_copy(
                xt_hbm.at[idx_v.at[cc * GB + j]],
                rows_v.at[b].at[pl.ds(j * 128, 128)],
                sems[b],
            )

    def drain(b):
        pltpu.make_async_copy(
            xt_hbm.at[pl.ds(0, PC * K)], rows_v.at[b], sems[b]
        ).wait()

    def compute(cc, b):
        def point_body(p, carry2):
            base = p * K
            vals = [
                [rows_v[b, base + k, pl.ds(F + f * 16, 16)] for f in range(FV)]
                for k in range(K)
            ]
            while len(vals) > 1:
                nxt = []
                for i in range(0, len(vals) - 1, 2):
                    nxt.append(
                        [jnp.maximum(a, c) for a, c in zip(vals[i], vals[i + 1])]
                    )
                if len(vals) % 2:
                    nxt.append(vals[-1])
                vals = nxt
            row = cc * PC + p
            for f in range(FV):
                out_v[row, pl.ds(f * 16, 16)] = vals[0][f]
            return carry2

        lax.fori_loop(0, PC, point_body, 0)

    fire(0, 0)

    def pair_body(t, carry):
        c2 = t * 2
        fire(c2 + 1, 1)
        drain(0)
        compute(c2, 0)

        @pl.when(c2 + 2 < NCHUNK)
        def _():
            fire(c2 + 2, 0)

        drain(1)
        compute(c2 + 1, 1)
        return carry

    lax.fori_loop(0, NCHUNK // 2, pair_body, 0)
    pltpu.sync_copy(out_v, agg_hbm.at[pl.ds(wid * PPW, PPW)])


_gather_max = pl.kernel(
    _gather_max_body,
    out_type=jax.ShapeDtypeStruct((NP, F), jnp.float32),
    mesh=_mesh,
    scratch_types=_SC_SCRATCH,
)


BN = 512
NBLK = NP // BN


def _mlp_body(x_ref, agg_ref, w1_ref, b1_ref, w2_ref, b2_ref, out_ref):
    aggb = agg_ref[...]
    h = jnp.maximum(
        jnp.dot(aggb, w1_ref[...], preferred_element_type=jnp.float32)
        + b1_ref[...],
        0.0,
    )
    f = (
        jnp.dot(h, w2_ref[...], preferred_element_type=jnp.float32)
        + b2_ref[...]
    )
    xb = x_ref[...]
    yfull = jnp.concatenate([xb[:, F:], xb[:, :F] + f], axis=1)
    ident = (
        lax.broadcasted_iota(jnp.int32, (BN, BN), 0)
        == lax.broadcasted_iota(jnp.int32, (BN, BN), 1)
    ).astype(jnp.float32)
    out_ref[...] = lax.dot_general(
        yfull,
        ident,
        (((0,), (0,)), ((), ())),
        preferred_element_type=jnp.float32,
    )


_mlp = pl.pallas_call(
    _mlp_body,
    grid=(NBLK,),
    in_specs=[
        pl.BlockSpec((BN, C), lambda i: (i, 0)),
        pl.BlockSpec((BN, F), lambda i: (i, 0)),
        pl.BlockSpec((F, F), lambda i: (0, 0)),
        pl.BlockSpec((1, F), lambda i: (0, 0)),
        pl.BlockSpec((F, F), lambda i: (0, 0)),
        pl.BlockSpec((1, F), lambda i: (0, 0)),
    ],
    out_specs=pl.BlockSpec((C, BN), lambda i: (0, i)),
    out_shape=jax.ShapeDtypeStruct((C, NP), jnp.float32),
)


def kernel(x, neigh_idx, W1, b1, W2, b2):
    x0 = x[0]
    x_pad = jnp.pad(x0, ((0, NP - N), (0, 0)))
    idx = neigh_idx[0].astype(jnp.int32)
    idx_pad = jnp.pad(idx, ((0, NP - N), (0, 0)))
    idx_r = idx_pad.reshape(NP * K // 128, 128)
    agg = _gather_max(x0, idx_r)
    outT = _mlp(x_pad, agg, W1, b1[None], W2, b2[None])
    return outT[None, :, :N]
